# Initial kernel scaffold; baseline (speedup 1.0000x reference)
import jax
import jax.numpy as jnp
from jax.experimental import pallas as pl


def kernel(obs, act, ag, g, W_mp, b_mp, W_p1a, b_p1a, W_p1b, b_p1b, W_p2a, b_p2a, W_p2b, b_p2b, W_r1a, b_r1a, W_r1b, b_r1b, W_r2a, b_r2a, W_r2b, b_r2b):
    raise NotImplementedError("write your pallas kernel here")


# speedup vs baseline: 4.1579x; 4.1579x over previous
import jax
import jax.numpy as jnp
from jax.experimental import pallas as pl
from jax.experimental.pallas import tpu as pltpu

_NB = 5
_DIM_BODY = 10
_DIM_OBJECT = 15
_EDGES = [[0, 1], [0, 2], [0, 3], [0, 4], [1, 0], [1, 2], [1, 3], [1, 4],
          [2, 0], [2, 1], [2, 3], [2, 4], [3, 0], [3, 1], [3, 2], [3, 4],
          [4, 0], [4, 1], [4, 2], [4, 3]]
_INCOMING = [[4, 8, 12, 16], [0, 9, 13, 17], [1, 5, 14, 18], [2, 6, 10, 19],
             [3, 7, 11, 15]]
_PRED_IDS = [[0, 10], [1, 11], [2, 12], [3, 13], [0, 14], [4, 15], [5, 16],
             [6, 17], [1, 18], [4, 19], [7, 20], [8, 21], [2, 22], [5, 23],
             [7, 24], [9, 25], [3, 26], [6, 27], [8, 28], [9, 29]]

_BT = 512


def _dot(a, b):
    return jax.lax.dot_general(a, b, (((1,), (0,)), ((), ())),
                               preferred_element_type=jnp.float32)


def _fused_kernel(obs_ref, act_ref, ag_ref, g_ref,
                  Wmp_ref, bmp_ref,
                  Wp1a_ref, bp1a_ref, Wp1b_ref, bp1b_ref,
                  Wp2a_ref, bp2a_ref, Wp2b_ref, bp2b_ref,
                  Wr1a_ref, br1a_ref, Wr1b_ref, br1b_ref,
                  Wr2a_ref, br2a_ref, Wr2b_ref, br2b_ref,
                  q1_ref, q2_ref):
    obs = obs_ref[...]
    act = act_ref[...]
    dg = g_ref[...] - ag_ref[...]
    body = obs[:, :_DIM_BODY]
    objs = [obs[:, _DIM_BODY + _DIM_OBJECT * j:
                   _DIM_BODY + _DIM_OBJECT * (j + 1)] for j in range(_NB)]

    Wmp = Wmp_ref[...]
    common_e = _dot(body, Wmp[0:10]) + _dot(act, Wmp[10:14]) + bmp_ref[...]
    Wsrc = Wmp[16:31]
    Wdst = Wmp[31:46]
    Ps = [_dot(o, Wsrc) for o in objs]
    Pd = [_dot(o, Wdst) for o in objs]
    wg0 = Wmp[14:15]
    wg1 = Wmp[15:16]
    ef = []
    for e in range(20):
        s, d = _EDGES[e]
        p0, p1 = _PRED_IDS[e]
        gpart = dg[:, p0:p0 + 1] * wg0 + dg[:, p1:p1 + 1] * wg1
        ef.append(jnp.maximum(common_e + Ps[s] + Pd[d] + gpart, 0.0))

    agg = [ef[a] + ef[b] + ef[c] + ef[d] for a, b, c, d in _INCOMING]

    Wp1a = Wp1a_ref[...]
    Wp2a = Wp2a_ref[...]
    Wp1b = Wp1b_ref[...]
    Wp2b = Wp2b_ref[...]
    bp1b = bp1b_ref[...]
    bp2b = bp2b_ref[...]
    c1 = _dot(act, Wp1a[0:4]) + _dot(body, Wp1a[4:14]) + bp1a_ref[...]
    c2 = _dot(act, Wp2a[0:4]) + _dot(body, Wp2a[4:14]) + bp2a_ref[...]
    Wo1 = Wp1a[14:29]
    Wa1 = Wp1a[29:157]
    Wo2 = Wp2a[14:29]
    Wa2 = Wp2a[29:157]
    o1 = None
    o2 = None
    for n in range(_NB):
        h1 = jnp.maximum(c1 + _dot(objs[n], Wo1) + _dot(agg[n], Wa1), 0.0)
        x1 = jnp.maximum(_dot(h1, Wp1b) + bp1b, 0.0)
        o1 = x1 if o1 is None else o1 + x1
        h2 = jnp.maximum(c2 + _dot(objs[n], Wo2) + _dot(agg[n], Wa2), 0.0)
        x2 = jnp.maximum(_dot(h2, Wp2b) + bp2b, 0.0)
        o2 = x2 if o2 is None else o2 + x2

    u1 = jnp.maximum(_dot(o1, Wr1a_ref[...]) + br1a_ref[...], 0.0)
    q1_ref[...] = _dot(u1, Wr1b_ref[...]) + br1b_ref[...]
    u2 = jnp.maximum(_dot(o2, Wr2a_ref[...]) + br2a_ref[...], 0.0)
    q2_ref[...] = _dot(u2, Wr2b_ref[...]) + br2b_ref[...]


def kernel(obs, act, ag, g, W_mp, b_mp, W_p1a, b_p1a, W_p1b, b_p1b,
           W_p2a, b_p2a, W_p2b, b_p2b, W_r1a, b_r1a, W_r1b, b_r1b,
           W_r2a, b_r2a, W_r2b, b_r2b):
    batch = obs.shape[0]
    grid = (batch // _BT,)

    def row_spec(cols):
        return pl.BlockSpec((_BT, cols), lambda i: (i, 0))

    def full_spec(shape):
        return pl.BlockSpec(shape, lambda i: (0,) * len(shape))

    biases = [b.reshape(1, -1) for b in
              (b_mp, b_p1a, b_p1b, b_p2a, b_p2b, b_r1a, b_r1b, b_r2a, b_r2b)]
    b_mp, b_p1a, b_p1b, b_p2a, b_p2b, b_r1a, b_r1b, b_r2a, b_r2b = biases

    args = (obs, act, ag, g,
            W_mp, b_mp, W_p1a, b_p1a, W_p1b, b_p1b,
            W_p2a, b_p2a, W_p2b, b_p2b,
            W_r1a, b_r1a, W_r1b, b_r1b, W_r2a, b_r2a, W_r2b, b_r2b)
    in_specs = [row_spec(obs.shape[1]), row_spec(act.shape[1]),
                row_spec(ag.shape[1]), row_spec(g.shape[1])]
    in_specs += [full_spec(a.shape) for a in args[4:]]

    q1, q2 = pl.pallas_call(
        _fused_kernel,
        grid=grid,
        in_specs=in_specs,
        out_specs=[pl.BlockSpec((_BT, 1), lambda i: (i, 0))] * 2,
        out_shape=[jax.ShapeDtypeStruct((batch, 1), jnp.float32)] * 2,
        compiler_params=pltpu.CompilerParams(
            dimension_semantics=("parallel",)),
    )(*args)
    return (q1, q2)


# speedup vs baseline: 5.8693x; 1.4116x over previous
import numpy as np
import jax
import jax.numpy as jnp
from jax.experimental import pallas as pl
from jax.experimental.pallas import tpu as pltpu

_NB = 5
_EDGES = [[0, 1], [0, 2], [0, 3], [0, 4], [1, 0], [1, 2], [1, 3], [1, 4],
          [2, 0], [2, 1], [2, 3], [2, 4], [3, 0], [3, 1], [3, 2], [3, 4],
          [4, 0], [4, 1], [4, 2], [4, 3]]
_INCOMING = [[4, 8, 12, 16], [0, 9, 13, 17], [1, 5, 14, 18], [2, 6, 10, 19],
             [3, 7, 11, 15]]
_PRED_IDS = [[0, 10], [1, 11], [2, 12], [3, 13], [0, 14], [4, 15], [5, 16],
             [6, 17], [1, 18], [4, 19], [7, 20], [8, 21], [2, 22], [5, 23],
             [7, 24], [9, 25], [3, 26], [6, 27], [8, 28], [9, 29]]

_BT = 512

_Z = 128


def _edge_onehot():
    P = np.zeros((20, _Z, 47), np.float32)
    for e, (s, d) in enumerate(_EDGES):
        p0, p1 = _PRED_IDS[e]
        for i in range(10):
            P[e, i, i] = 1.0
        for i in range(4):
            P[e, 10 + i, 10 + i] = 1.0
        P[e, 14 + p0, 14] = 1.0
        P[e, 14 + p1, 15] = 1.0
        for i in range(15):
            P[e, 44 + 15 * s + i, 16 + i] = 1.0
            P[e, 44 + 15 * d + i, 31 + i] = 1.0
        P[e, 119, 46] = 1.0
    return P


def _phi_onehot():
    Q = np.zeros((_NB, _Z, 30), np.float32)
    for n in range(_NB):
        for i in range(4):
            Q[n, 10 + i, i] = 1.0
        for i in range(10):
            Q[n, i, 4 + i] = 1.0
        for i in range(15):
            Q[n, 44 + 15 * n + i, 14 + i] = 1.0
        Q[n, 119, 29] = 1.0
    return Q


_P_EDGE = _edge_onehot()
_Q_PHI = _phi_onehot()


def _dot(a, b):
    return jax.lax.dot_general(a, b, (((1,), (0,)), ((), ())),
                               preferred_element_type=jnp.float32)


def _fused_kernel(obs_ref, act_ref, ag_ref, g_ref,
                  We_ref, Wz_ref, Wagg_ref,
                  Wp1b_ref, bp1b_ref, Wp2b_ref, bp2b_ref,
                  Wr1a_ref, br1a_ref, Wr1b_ref, br1b_ref,
                  Wr2a_ref, br2a_ref, Wr2b_ref, br2b_ref,
                  q1_ref, q2_ref):
    obs = obs_ref[...]
    act = act_ref[...]
    dg = g_ref[...] - ag_ref[...]
    bt = obs.shape[0]
    z = jnp.concatenate(
        [obs[:, :10], act, dg, obs[:, 10:],
         jnp.full((bt, 1), 1.0, jnp.float32),
         jnp.zeros((bt, _Z - 120), jnp.float32)], axis=1)

    ef = jnp.maximum(_dot(z, We_ref[...]), 0.0)
    agg = [ef[:, 128 * a:128 * a + 128] + ef[:, 128 * b:128 * b + 128]
           + ef[:, 128 * c:128 * c + 128] + ef[:, 128 * d:128 * d + 128]
           for a, b, c, d in _INCOMING]

    hpre = _dot(z, Wz_ref[...])
    Wagg = Wagg_ref[...]
    Wp1b = Wp1b_ref[...]
    Wp2b = Wp2b_ref[...]
    bp1b = bp1b_ref[...]
    bp2b = bp2b_ref[...]
    o1 = None
    o2 = None
    for n in range(_NB):
        h = jnp.maximum(hpre[:, 512 * n:512 * n + 512] + _dot(agg[n], Wagg),
                        0.0)
        x1 = jnp.maximum(_dot(h[:, :256], Wp1b) + bp1b, 0.0)
        x2 = jnp.maximum(_dot(h[:, 256:], Wp2b) + bp2b, 0.0)
        o1 = x1 if o1 is None else o1 + x1
        o2 = x2 if o2 is None else o2 + x2

    u1 = jnp.maximum(_dot(o1, Wr1a_ref[...]) + br1a_ref[...], 0.0)
    q1_ref[...] = _dot(u1, Wr1b_ref[...]) + br1b_ref[...]
    u2 = jnp.maximum(_dot(o2, Wr2a_ref[...]) + br2a_ref[...], 0.0)
    q2_ref[...] = _dot(u2, Wr2b_ref[...]) + br2b_ref[...]


def kernel(obs, act, ag, g, W_mp, b_mp, W_p1a, b_p1a, W_p1b, b_p1b,
           W_p2a, b_p2a, W_p2b, b_p2b, W_r1a, b_r1a, W_r1b, b_r1b,
           W_r2a, b_r2a, W_r2b, b_r2b):
    batch = obs.shape[0]
    grid = (batch // _BT,)

    We_aug = jnp.concatenate([W_mp, b_mp[None]], axis=0)
    We = jnp.einsum('erj,jk->rek', _P_EDGE, We_aug).reshape(_Z, 20 * 128)
    W1_aug = jnp.concatenate([W_p1a[:29], b_p1a[None]], axis=0)
    W2_aug = jnp.concatenate([W_p2a[:29], b_p2a[None]], axis=0)
    Wz1 = jnp.einsum('nrj,jk->rnk', _Q_PHI, W1_aug)
    Wz2 = jnp.einsum('nrj,jk->rnk', _Q_PHI, W2_aug)
    Wz = jnp.concatenate([Wz1, Wz2], axis=2).reshape(_Z, _NB * 512)
    Wagg = jnp.concatenate([W_p1a[29:], W_p2a[29:]], axis=1)

    args = (obs, act, ag, g, We, Wz, Wagg,
            W_p1b, b_p1b.reshape(1, -1), W_p2b, b_p2b.reshape(1, -1),
            W_r1a, b_r1a.reshape(1, -1), W_r1b, b_r1b.reshape(1, -1),
            W_r2a, b_r2a.reshape(1, -1), W_r2b, b_r2b.reshape(1, -1))

    def row_spec(cols):
        return pl.BlockSpec((_BT, cols), lambda i: (i, 0))

    in_specs = [row_spec(obs.shape[1]), row_spec(act.shape[1]),
                row_spec(ag.shape[1]), row_spec(g.shape[1])]
    in_specs += [pl.BlockSpec(a.shape, lambda i: (0, 0)) for a in args[4:]]

    q1, q2 = pl.pallas_call(
        _fused_kernel,
        grid=grid,
        in_specs=in_specs,
        out_specs=[pl.BlockSpec((_BT, 1), lambda i: (i, 0))] * 2,
        out_shape=[jax.ShapeDtypeStruct((batch, 1), jnp.float32)] * 2,
        compiler_params=pltpu.CompilerParams(
            dimension_semantics=("parallel",)),
    )(*args)
    return (q1, q2)


# speedup vs baseline: 6.0255x; 1.0266x over previous
import numpy as np
import jax
import jax.numpy as jnp
from jax.experimental import pallas as pl
from jax.experimental.pallas import tpu as pltpu

_NB = 5
_EDGES = [[0, 1], [0, 2], [0, 3], [0, 4], [1, 0], [1, 2], [1, 3], [1, 4],
          [2, 0], [2, 1], [2, 3], [2, 4], [3, 0], [3, 1], [3, 2], [3, 4],
          [4, 0], [4, 1], [4, 2], [4, 3]]
_INCOMING = [[4, 8, 12, 16], [0, 9, 13, 17], [1, 5, 14, 18], [2, 6, 10, 19],
             [3, 7, 11, 15]]
_PRED_IDS = [[0, 10], [1, 11], [2, 12], [3, 13], [0, 14], [4, 15], [5, 16],
             [6, 17], [1, 18], [4, 19], [7, 20], [8, 21], [2, 22], [5, 23],
             [7, 24], [9, 25], [3, 26], [6, 27], [8, 28], [9, 29]]

_BT = 1024

_Z = 128


def _edge_onehot():
    P = np.zeros((20, _Z, 47), np.float32)
    for e, (s, d) in enumerate(_EDGES):
        p0, p1 = _PRED_IDS[e]
        for i in range(10):
            P[e, i, i] = 1.0
        for i in range(4):
            P[e, 10 + i, 10 + i] = 1.0
        P[e, 14 + p0, 14] = 1.0
        P[e, 14 + p1, 15] = 1.0
        for i in range(15):
            P[e, 44 + 15 * s + i, 16 + i] = 1.0
            P[e, 44 + 15 * d + i, 31 + i] = 1.0
        P[e, 119, 46] = 1.0
    return P


def _phi_onehot():
    Q = np.zeros((_NB, _Z, 30), np.float32)
    for n in range(_NB):
        for i in range(4):
            Q[n, 10 + i, i] = 1.0
        for i in range(10):
            Q[n, i, 4 + i] = 1.0
        for i in range(15):
            Q[n, 44 + 15 * n + i, 14 + i] = 1.0
        Q[n, 119, 29] = 1.0
    return Q


_P_EDGE = _edge_onehot()
_Q_PHI = _phi_onehot()


def _dot(a, b):
    return jax.lax.dot_general(a, b, (((1,), (0,)), ((), ())),
                               preferred_element_type=jnp.float32)


def _fused_kernel(obs_ref, act_ref, ag_ref, g_ref,
                  We_ref, Wz_ref, Wagg_ref,
                  Wp1b_ref, bp1b_ref, Wp2b_ref, bp2b_ref,
                  Wr1a_ref, br1a_ref, Wr1b_ref, br1b_ref,
                  Wr2a_ref, br2a_ref, Wr2b_ref, br2b_ref,
                  q1_ref, q2_ref):
    obs = obs_ref[...]
    act = act_ref[...]
    dg = g_ref[...] - ag_ref[...]
    bt = obs.shape[0]
    z = jnp.concatenate(
        [obs[:, :10], act, dg, obs[:, 10:],
         jnp.full((bt, 1), 1.0, jnp.float32),
         jnp.zeros((bt, _Z - 120), jnp.float32)], axis=1)

    ef = jnp.maximum(_dot(z, We_ref[...]), 0.0)
    agg = [ef[:, 128 * a:128 * a + 128] + ef[:, 128 * b:128 * b + 128]
           + ef[:, 128 * c:128 * c + 128] + ef[:, 128 * d:128 * d + 128]
           for a, b, c, d in _INCOMING]

    hpre = _dot(z, Wz_ref[...])
    Wagg = Wagg_ref[...]
    Wp1b = Wp1b_ref[...]
    Wp2b = Wp2b_ref[...]
    bp1b = bp1b_ref[...]
    bp2b = bp2b_ref[...]
    o1 = None
    o2 = None
    for n in range(_NB):
        h = jnp.maximum(hpre[:, 512 * n:512 * n + 512] + _dot(agg[n], Wagg),
                        0.0)
        x1 = jnp.maximum(_dot(h[:, :256], Wp1b) + bp1b, 0.0)
        x2 = jnp.maximum(_dot(h[:, 256:], Wp2b) + bp2b, 0.0)
        o1 = x1 if o1 is None else o1 + x1
        o2 = x2 if o2 is None else o2 + x2

    u1 = jnp.maximum(_dot(o1, Wr1a_ref[...]) + br1a_ref[...], 0.0)
    q1_ref[...] = _dot(u1, Wr1b_ref[...]) + br1b_ref[...]
    u2 = jnp.maximum(_dot(o2, Wr2a_ref[...]) + br2a_ref[...], 0.0)
    q2_ref[...] = _dot(u2, Wr2b_ref[...]) + br2b_ref[...]


def kernel(obs, act, ag, g, W_mp, b_mp, W_p1a, b_p1a, W_p1b, b_p1b,
           W_p2a, b_p2a, W_p2b, b_p2b, W_r1a, b_r1a, W_r1b, b_r1b,
           W_r2a, b_r2a, W_r2b, b_r2b):
    batch = obs.shape[0]
    grid = (batch // _BT,)

    We_aug = jnp.concatenate([W_mp, b_mp[None]], axis=0)
    We = jnp.einsum('erj,jk->rek', _P_EDGE, We_aug).reshape(_Z, 20 * 128)
    W1_aug = jnp.concatenate([W_p1a[:29], b_p1a[None]], axis=0)
    W2_aug = jnp.concatenate([W_p2a[:29], b_p2a[None]], axis=0)
    Wz1 = jnp.einsum('nrj,jk->rnk', _Q_PHI, W1_aug)
    Wz2 = jnp.einsum('nrj,jk->rnk', _Q_PHI, W2_aug)
    Wz = jnp.concatenate([Wz1, Wz2], axis=2).reshape(_Z, _NB * 512)
    Wagg = jnp.concatenate([W_p1a[29:], W_p2a[29:]], axis=1)

    args = (obs, act, ag, g, We, Wz, Wagg,
            W_p1b, b_p1b.reshape(1, -1), W_p2b, b_p2b.reshape(1, -1),
            W_r1a, b_r1a.reshape(1, -1), W_r1b, b_r1b.reshape(1, -1),
            W_r2a, b_r2a.reshape(1, -1), W_r2b, b_r2b.reshape(1, -1))

    def row_spec(cols):
        return pl.BlockSpec((_BT, cols), lambda i: (i, 0))

    in_specs = [row_spec(obs.shape[1]), row_spec(act.shape[1]),
                row_spec(ag.shape[1]), row_spec(g.shape[1])]
    in_specs += [pl.BlockSpec(a.shape, lambda i: (0, 0)) for a in args[4:]]

    q1, q2 = pl.pallas_call(
        _fused_kernel,
        grid=grid,
        in_specs=in_specs,
        out_specs=[pl.BlockSpec((_BT, 1), lambda i: (i, 0))] * 2,
        out_shape=[jax.ShapeDtypeStruct((batch, 1), jnp.float32)] * 2,
        compiler_params=pltpu.CompilerParams(
            dimension_semantics=("parallel",)),
    )(*args)
    return (q1, q2)


# speedup vs baseline: 6.1084x; 1.0138x over previous
import numpy as np
import jax
import jax.numpy as jnp
from jax.experimental import pallas as pl
from jax.experimental.pallas import tpu as pltpu

_NB = 5
_EDGES = [[0, 1], [0, 2], [0, 3], [0, 4], [1, 0], [1, 2], [1, 3], [1, 4],
          [2, 0], [2, 1], [2, 3], [2, 4], [3, 0], [3, 1], [3, 2], [3, 4],
          [4, 0], [4, 1], [4, 2], [4, 3]]
_INCOMING = [[4, 8, 12, 16], [0, 9, 13, 17], [1, 5, 14, 18], [2, 6, 10, 19],
             [3, 7, 11, 15]]
_PRED_IDS = [[0, 10], [1, 11], [2, 12], [3, 13], [0, 14], [4, 15], [5, 16],
             [6, 17], [1, 18], [4, 19], [7, 20], [8, 21], [2, 22], [5, 23],
             [7, 24], [9, 25], [3, 26], [6, 27], [8, 28], [9, 29]]

_BT = 2048

_Z = 128


def _edge_onehot():
    P = np.zeros((20, _Z, 47), np.float32)
    for e, (s, d) in enumerate(_EDGES):
        p0, p1 = _PRED_IDS[e]
        for i in range(10):
            P[e, i, i] = 1.0
        for i in range(4):
            P[e, 10 + i, 10 + i] = 1.0
        P[e, 14 + p0, 14] = 1.0
        P[e, 14 + p1, 15] = 1.0
        for i in range(15):
            P[e, 44 + 15 * s + i, 16 + i] = 1.0
            P[e, 44 + 15 * d + i, 31 + i] = 1.0
        P[e, 119, 46] = 1.0
    return P


def _phi_onehot():
    Q = np.zeros((_NB, _Z, 30), np.float32)
    for n in range(_NB):
        for i in range(4):
            Q[n, 10 + i, i] = 1.0
        for i in range(10):
            Q[n, i, 4 + i] = 1.0
        for i in range(15):
            Q[n, 44 + 15 * n + i, 14 + i] = 1.0
        Q[n, 119, 29] = 1.0
    return Q


_P_EDGE = _edge_onehot()
_Q_PHI = _phi_onehot()


def _dot(a, b):
    return jax.lax.dot_general(a, b, (((1,), (0,)), ((), ())),
                               preferred_element_type=jnp.float32)


def _fused_kernel(obs_ref, act_ref, ag_ref, g_ref,
                  We_ref, Wz_ref, Wagg_ref,
                  Wp1b_ref, bp1b_ref, Wp2b_ref, bp2b_ref,
                  Wr1a_ref, br1a_ref, Wr1b_ref, br1b_ref,
                  Wr2a_ref, br2a_ref, Wr2b_ref, br2b_ref,
                  q1_ref, q2_ref):
    obs = obs_ref[...]
    act = act_ref[...]
    dg = g_ref[...] - ag_ref[...]
    bt = obs.shape[0]
    z = jnp.concatenate(
        [obs[:, :10], act, dg, obs[:, 10:],
         jnp.full((bt, 1), 1.0, jnp.float32),
         jnp.zeros((bt, _Z - 120), jnp.float32)], axis=1)

    ef = jnp.maximum(_dot(z, We_ref[...]), 0.0)
    agg = [ef[:, 128 * a:128 * a + 128] + ef[:, 128 * b:128 * b + 128]
           + ef[:, 128 * c:128 * c + 128] + ef[:, 128 * d:128 * d + 128]
           for a, b, c, d in _INCOMING]

    hpre = _dot(z, Wz_ref[...])
    Wagg = Wagg_ref[...]
    Wp1b = Wp1b_ref[...]
    Wp2b = Wp2b_ref[...]
    bp1b = bp1b_ref[...]
    bp2b = bp2b_ref[...]
    o1 = None
    o2 = None
    for n in range(_NB):
        h = jnp.maximum(hpre[:, 512 * n:512 * n + 512] + _dot(agg[n], Wagg),
                        0.0)
        x1 = jnp.maximum(_dot(h[:, :256], Wp1b) + bp1b, 0.0)
        x2 = jnp.maximum(_dot(h[:, 256:], Wp2b) + bp2b, 0.0)
        o1 = x1 if o1 is None else o1 + x1
        o2 = x2 if o2 is None else o2 + x2

    u1 = jnp.maximum(_dot(o1, Wr1a_ref[...]) + br1a_ref[...], 0.0)
    q1_ref[...] = _dot(u1, Wr1b_ref[...]) + br1b_ref[...]
    u2 = jnp.maximum(_dot(o2, Wr2a_ref[...]) + br2a_ref[...], 0.0)
    q2_ref[...] = _dot(u2, Wr2b_ref[...]) + br2b_ref[...]


def kernel(obs, act, ag, g, W_mp, b_mp, W_p1a, b_p1a, W_p1b, b_p1b,
           W_p2a, b_p2a, W_p2b, b_p2b, W_r1a, b_r1a, W_r1b, b_r1b,
           W_r2a, b_r2a, W_r2b, b_r2b):
    batch = obs.shape[0]
    grid = (batch // _BT,)

    We_aug = jnp.concatenate([W_mp, b_mp[None]], axis=0)
    We = jnp.einsum('erj,jk->rek', _P_EDGE, We_aug).reshape(_Z, 20 * 128)
    W1_aug = jnp.concatenate([W_p1a[:29], b_p1a[None]], axis=0)
    W2_aug = jnp.concatenate([W_p2a[:29], b_p2a[None]], axis=0)
    Wz1 = jnp.einsum('nrj,jk->rnk', _Q_PHI, W1_aug)
    Wz2 = jnp.einsum('nrj,jk->rnk', _Q_PHI, W2_aug)
    Wz = jnp.concatenate([Wz1, Wz2], axis=2).reshape(_Z, _NB * 512)
    Wagg = jnp.concatenate([W_p1a[29:], W_p2a[29:]], axis=1)

    args = (obs, act, ag, g, We, Wz, Wagg,
            W_p1b, b_p1b.reshape(1, -1), W_p2b, b_p2b.reshape(1, -1),
            W_r1a, b_r1a.reshape(1, -1), W_r1b, b_r1b.reshape(1, -1),
            W_r2a, b_r2a.reshape(1, -1), W_r2b, b_r2b.reshape(1, -1))

    def row_spec(cols):
        return pl.BlockSpec((_BT, cols), lambda i: (i, 0))

    in_specs = [row_spec(obs.shape[1]), row_spec(act.shape[1]),
                row_spec(ag.shape[1]), row_spec(g.shape[1])]
    in_specs += [pl.BlockSpec(a.shape, lambda i: (0, 0)) for a in args[4:]]

    q1, q2 = pl.pallas_call(
        _fused_kernel,
        grid=grid,
        in_specs=in_specs,
        out_specs=[pl.BlockSpec((_BT, 1), lambda i: (i, 0))] * 2,
        out_shape=[jax.ShapeDtypeStruct((batch, 1), jnp.float32)] * 2,
        compiler_params=pltpu.CompilerParams(
            dimension_semantics=("parallel",)),
    )(*args)
    return (q1, q2)
